# baseline (speedup 1.0000x reference)
import functools

import jax
import jax.numpy as jnp
from jax import lax
from jax.experimental import pallas as pl
from jax.experimental.pallas import tpu as pltpu
from jax.experimental.pallas import tpu_sc as plsc


def kernel(x, emb_table):
    B0, S = x.shape
    V, D = emb_table.shape
    B = B0 * S

    info = plsc.get_sparse_core_info()
    NC, NS = info.num_cores, info.num_subcores
    NW = NC * NS

    K = 128
    R = 5
    b_per_w = B // NW
    n_chunks = b_per_w // K

    idx2d = x.reshape(B // K, K).astype(jnp.int32)

    mesh = plsc.VectorSubcoreMesh(core_axis_name="c", subcore_axis_name="s")

    @functools.partial(
        pl.kernel,
        mesh=mesh,
        out_type=jax.ShapeDtypeStruct((B, D), jnp.float32),
        scratch_types=(
            [pltpu.VMEM((n_chunks, K), jnp.int32)]
            + [pltpu.VMEM((K, D), jnp.float32) for _ in range(R)]
            + [pltpu.SemaphoreType.DMA for _ in range(2 * R)]
        ),
    )
    def gather_kernel(table_hbm, idx_hbm, out_hbm, idx_all, *bufs_and_sems):
        rows = bufs_and_sems[:R]
        gsem = bufs_and_sems[R:2 * R]
        ssem = bufs_and_sems[2 * R:]

        wid = lax.axis_index("s") * NC + lax.axis_index("c")
        idx_row0 = wid * n_chunks
        out_base = wid * b_per_w

        pltpu.sync_copy(idx_hbm.at[pl.ds(idx_row0, n_chunks)], idx_all)

        def fire_gather(i, r):
            pltpu.async_copy(table_hbm.at[idx_all.at[i]], rows[r], gsem[r])

        def wait_gather(r):
            pltpu.make_async_copy(table_hbm.at[idx_all.at[0]], rows[r],
                                  gsem[r]).wait()

        def fire_scatter(i, r):
            pltpu.async_copy(rows[r], out_hbm.at[pl.ds(out_base + i * K, K)],
                             ssem[r])

        def wait_scatter(r):
            pltpu.make_async_copy(rows[r], out_hbm.at[pl.ds(out_base, K)],
                                  ssem[r]).wait()

        for i in range(R - 1):
            fire_gather(i, i)
        for r in range(R):
            wait_gather(r)
            fire_scatter(r, r)
            rp = (r + R - 1) % R
            if r > 0:
                wait_scatter(rp)
            fire_gather(r + R - 1, rp)

        def body(t, carry):
            for r in range(R):
                i = R * t + r
                wait_gather(r)
                fire_scatter(i, r)
                rp = (r + R - 1) % R
                wait_scatter(rp)
                fire_gather(i + R - 1, rp)
            return carry

        lax.fori_loop(1, n_chunks // R - 1, body, 0)

        base = n_chunks - R
        for r in range(R):
            wait_gather(r)
            fire_scatter(base + r, r)
            if r == 0:
                rp = R - 1
                wait_scatter(rp)
                fire_gather(n_chunks - 1, rp)
        for r in range(R):
            wait_scatter(r)

    out = gather_kernel(emb_table, idx2d)
    return out.reshape(B0, S, D)
